# Initial kernel scaffold; baseline (speedup 1.0000x reference)
import jax
import jax.numpy as jnp
from jax.experimental import pallas as pl


def kernel(x, batch, W1, b1, W2, b2):
    raise NotImplementedError("write your pallas kernel here")


# speedup vs baseline: 3.9954x; 3.9954x over previous
import jax
import jax.numpy as jnp
from jax.experimental import pallas as pl
from jax.experimental.pallas import tpu as pltpu

SEG = 64
R = 2000

_NEG_INF = float("-inf")


def _stage1_kernel(x_ref, b_ref, w1_ref, b1_ref, w2_ref, b2_ref,
                   pacc_ref, pm_ref, ps_ref):
    xb = x_ref[...]
    h = jnp.tanh(
        jnp.dot(xb, w1_ref[...], preferred_element_type=jnp.float32)
        + b1_ref[...])
    logit = (jnp.sum(h * w2_ref[...], axis=1, keepdims=True)
             + b2_ref[...])

    bb = b_ref[...]
    seg_ids = jax.lax.broadcasted_iota(jnp.int32, (bb.shape[0], SEG), 1)
    onehot = bb == seg_ids

    m_blk = jnp.max(jnp.where(onehot, logit, _NEG_INF), axis=0)
    m_row = jnp.sum(jnp.where(onehot, m_blk[None, :], 0.0),
                    axis=1, keepdims=True)
    p = jnp.exp(logit - m_row)
    woh = jnp.where(onehot, p, 0.0)

    ps = jnp.sum(woh, axis=0)
    pacc = jax.lax.dot_general(
        woh, xb, (((0,), (0,)), ((), ())),
        preferred_element_type=jnp.float32)

    pacc_ref[...] = pacc[None]
    pm_ref[...] = m_blk.reshape(1, 1, SEG)
    ps_ref[...] = ps.reshape(1, 1, SEG)


def _stage2_kernel(pacc_ref, pm_ref, ps_ref, out_ref,
                   acc_ref, m_ref, s_ref):
    i = pl.program_id(0)
    nblk = pl.num_programs(0)

    @pl.when(i == 0)
    def _init():
        acc_ref[...] = jnp.zeros_like(acc_ref)
        m_ref[...] = jnp.full_like(m_ref, _NEG_INF)
        s_ref[...] = jnp.zeros_like(s_ref)

    m_old = m_ref[...]
    pmv = pm_ref[0]
    psv = ps_ref[0]

    m_new = jnp.maximum(m_old, pmv)
    sc_old = jnp.where(m_old == _NEG_INF, 0.0, jnp.exp(m_old - m_new))
    sc_new = jnp.where(pmv == _NEG_INF, 0.0, jnp.exp(pmv - m_new))

    s_val = s_ref[...] * sc_old + psv * sc_new
    acc_val = (acc_ref[...] * sc_old.reshape(SEG, 1)
               + pacc_ref[0] * sc_new.reshape(SEG, 1))

    m_ref[...] = m_new
    s_ref[...] = s_val
    acc_ref[...] = acc_val

    @pl.when(i == nblk - 1)
    def _finish():
        out_ref[...] = acc_val / (s_val.reshape(SEG, 1) + 1e-8)


def kernel(x, batch, W1, b1, W2, b2):
    N, D = x.shape
    H = W1.shape[1]
    nblk = N // R
    assert N % R == 0

    batch2 = batch.reshape(N, 1)
    b1r = b1.reshape(1, H)
    w2r = W2.reshape(1, H)
    b2r = b2.reshape(1, 1)

    pacc, pm, ps = pl.pallas_call(
        _stage1_kernel,
        grid=(nblk,),
        in_specs=[
            pl.BlockSpec((R, D), lambda i: (i, 0)),
            pl.BlockSpec((R, 1), lambda i: (i, 0)),
            pl.BlockSpec((D, H), lambda i: (0, 0)),
            pl.BlockSpec((1, H), lambda i: (0, 0)),
            pl.BlockSpec((1, H), lambda i: (0, 0)),
            pl.BlockSpec((1, 1), lambda i: (0, 0)),
        ],
        out_specs=[
            pl.BlockSpec((1, SEG, D), lambda i: (i, 0, 0)),
            pl.BlockSpec((1, 1, SEG), lambda i: (i, 0, 0)),
            pl.BlockSpec((1, 1, SEG), lambda i: (i, 0, 0)),
        ],
        out_shape=[
            jax.ShapeDtypeStruct((nblk, SEG, D), jnp.float32),
            jax.ShapeDtypeStruct((nblk, 1, SEG), jnp.float32),
            jax.ShapeDtypeStruct((nblk, 1, SEG), jnp.float32),
        ],
        compiler_params=pltpu.CompilerParams(
            dimension_semantics=("parallel",)),
    )(x, batch2, W1, b1r, w2r, b2r)

    out = pl.pallas_call(
        _stage2_kernel,
        grid=(nblk,),
        in_specs=[
            pl.BlockSpec((1, SEG, D), lambda i: (i, 0, 0)),
            pl.BlockSpec((1, 1, SEG), lambda i: (i, 0, 0)),
            pl.BlockSpec((1, 1, SEG), lambda i: (i, 0, 0)),
        ],
        out_specs=pl.BlockSpec((SEG, D), lambda i: (0, 0)),
        out_shape=jax.ShapeDtypeStruct((SEG, D), jnp.float32),
        scratch_shapes=[
            pltpu.VMEM((SEG, D), jnp.float32),
            pltpu.VMEM((1, SEG), jnp.float32),
            pltpu.VMEM((1, SEG), jnp.float32),
        ],
        compiler_params=pltpu.CompilerParams(
            dimension_semantics=("arbitrary",)),
    )(pacc, pm, ps)

    return out


# speedup vs baseline: 5.8860x; 1.4732x over previous
import jax
import jax.numpy as jnp
from jax.experimental import pallas as pl
from jax.experimental.pallas import tpu as pltpu

SEG = 64
R = 4000

_NEG_INF = float("-inf")


def _stage1_kernel(x_ref, b_ref, w1_ref, b1_ref, w2_ref, b2_ref,
                   pacc_ref, pm_ref, ps_ref):
    xb = x_ref[...]
    h = jnp.tanh(
        jnp.dot(xb, w1_ref[...], preferred_element_type=jnp.float32)
        + b1_ref[...])
    logit = (jnp.sum(h * w2_ref[...], axis=1, keepdims=True)
             + b2_ref[...])
    logit_t = logit.reshape(1, -1)

    bbt = b_ref[0]
    seg_ids = jax.lax.broadcasted_iota(jnp.int32, (SEG, logit_t.shape[1]), 0)
    lmask = jnp.where(bbt == seg_ids, logit_t, _NEG_INF)
    m_blk = jnp.max(lmask, axis=1, keepdims=True)
    woh = jnp.exp(lmask - jnp.maximum(m_blk, -1e30))

    ps = jnp.sum(woh, axis=1, keepdims=True)
    pacc = jnp.dot(woh, xb, preferred_element_type=jnp.float32)

    pacc_ref[...] = pacc[None]
    pm_ref[...] = m_blk[None]
    ps_ref[...] = ps[None]


def _stage2_kernel(pacc_ref, pm_ref, ps_ref, out_ref,
                   acc_ref, m_ref, s_ref):
    i = pl.program_id(0)
    nblk = pl.num_programs(0)

    @pl.when(i == 0)
    def _init():
        acc_ref[...] = jnp.zeros_like(acc_ref)
        m_ref[...] = jnp.full_like(m_ref, _NEG_INF)
        s_ref[...] = jnp.zeros_like(s_ref)

    m_old = m_ref[...]
    pmv = pm_ref[0]
    psv = ps_ref[0]

    m_new = jnp.maximum(m_old, pmv)
    sc_old = jnp.where(m_old == _NEG_INF, 0.0, jnp.exp(m_old - m_new))
    sc_new = jnp.where(pmv == _NEG_INF, 0.0, jnp.exp(pmv - m_new))

    s_val = s_ref[...] * sc_old + psv * sc_new
    acc_val = acc_ref[...] * sc_old + pacc_ref[0] * sc_new

    m_ref[...] = m_new
    s_ref[...] = s_val
    acc_ref[...] = acc_val

    @pl.when(i == nblk - 1)
    def _finish():
        out_ref[...] = acc_val / (s_val + 1e-8)


def kernel(x, batch, W1, b1, W2, b2):
    N, D = x.shape
    H = W1.shape[1]
    nblk = N // R
    assert N % R == 0

    batch3 = batch.reshape(nblk, 1, R)
    b1r = b1.reshape(1, H)
    w2r = W2.reshape(1, H)
    b2r = b2.reshape(1, 1)

    pacc, pm, ps = pl.pallas_call(
        _stage1_kernel,
        grid=(nblk,),
        in_specs=[
            pl.BlockSpec((R, D), lambda i: (i, 0)),
            pl.BlockSpec((1, 1, R), lambda i: (i, 0, 0)),
            pl.BlockSpec((D, H), lambda i: (0, 0)),
            pl.BlockSpec((1, H), lambda i: (0, 0)),
            pl.BlockSpec((1, H), lambda i: (0, 0)),
            pl.BlockSpec((1, 1), lambda i: (0, 0)),
        ],
        out_specs=[
            pl.BlockSpec((1, SEG, D), lambda i: (i, 0, 0)),
            pl.BlockSpec((1, SEG, 1), lambda i: (i, 0, 0)),
            pl.BlockSpec((1, SEG, 1), lambda i: (i, 0, 0)),
        ],
        out_shape=[
            jax.ShapeDtypeStruct((nblk, SEG, D), jnp.float32),
            jax.ShapeDtypeStruct((nblk, SEG, 1), jnp.float32),
            jax.ShapeDtypeStruct((nblk, SEG, 1), jnp.float32),
        ],
        compiler_params=pltpu.CompilerParams(
            dimension_semantics=("parallel",)),
    )(x, batch3, W1, b1r, w2r, b2r)

    out = pl.pallas_call(
        _stage2_kernel,
        grid=(nblk,),
        in_specs=[
            pl.BlockSpec((1, SEG, D), lambda i: (i, 0, 0)),
            pl.BlockSpec((1, SEG, 1), lambda i: (i, 0, 0)),
            pl.BlockSpec((1, SEG, 1), lambda i: (i, 0, 0)),
        ],
        out_specs=pl.BlockSpec((SEG, D), lambda i: (0, 0)),
        out_shape=jax.ShapeDtypeStruct((SEG, D), jnp.float32),
        scratch_shapes=[
            pltpu.VMEM((SEG, D), jnp.float32),
            pltpu.VMEM((SEG, 1), jnp.float32),
            pltpu.VMEM((SEG, 1), jnp.float32),
        ],
        compiler_params=pltpu.CompilerParams(
            dimension_semantics=("arbitrary",)),
    )(pacc, pm, ps)

    return out


# speedup vs baseline: 10.2972x; 1.7494x over previous
import jax
import jax.numpy as jnp
from jax.experimental import pallas as pl
from jax.experimental.pallas import tpu as pltpu

SEG = 64
R = 4000

_NEG_INF = float("-inf")


def _stage1_kernel(x_ref, b_ref, w1_ref, b1_ref, w2_ref, b2_ref,
                   pacc_ref, pm_ref, ps_ref):
    xb = x_ref[...]
    h = jnp.tanh(
        jnp.dot(xb.astype(jnp.bfloat16), w1_ref[...],
                preferred_element_type=jnp.float32)
        + b1_ref[...])
    logit = (jnp.dot(h, w2_ref[...], preferred_element_type=jnp.float32)
             + b2_ref[...])
    logit_t = logit.reshape(1, -1)

    bbt = b_ref[0]
    seg_ids = jax.lax.broadcasted_iota(jnp.int32, (SEG, logit_t.shape[1]), 0)
    lmask = jnp.where(bbt == seg_ids, logit_t, _NEG_INF)
    m_blk = jnp.max(lmask, axis=1, keepdims=True)
    woh = jnp.exp(lmask - jnp.maximum(m_blk, -1e30))

    ps = jnp.sum(woh, axis=1, keepdims=True)
    pacc = jnp.dot(woh, xb, preferred_element_type=jnp.float32)

    pacc_ref[...] = pacc[None]
    pm_ref[...] = m_blk[None]
    ps_ref[...] = ps[None]


def _stage2_kernel(pacc_ref, pm_ref, ps_ref, out_ref,
                   acc_ref, m_ref, s_ref):
    i = pl.program_id(0)
    nblk = pl.num_programs(0)

    @pl.when(i == 0)
    def _init():
        acc_ref[...] = jnp.zeros_like(acc_ref)
        m_ref[...] = jnp.full_like(m_ref, _NEG_INF)
        s_ref[...] = jnp.zeros_like(s_ref)

    m_old = m_ref[...]
    pmv = pm_ref[0]
    psv = ps_ref[0]

    m_new = jnp.maximum(m_old, pmv)
    sc_old = jnp.where(m_old == _NEG_INF, 0.0, jnp.exp(m_old - m_new))
    sc_new = jnp.where(pmv == _NEG_INF, 0.0, jnp.exp(pmv - m_new))

    s_val = s_ref[...] * sc_old + psv * sc_new
    acc_val = acc_ref[...] * sc_old + pacc_ref[0] * sc_new

    m_ref[...] = m_new
    s_ref[...] = s_val
    acc_ref[...] = acc_val

    @pl.when(i == nblk - 1)
    def _finish():
        out_ref[...] = acc_val / (s_val + 1e-8)


def kernel(x, batch, W1, b1, W2, b2):
    N, D = x.shape
    H = W1.shape[1]
    nblk = N // R
    assert N % R == 0

    batch3 = batch.reshape(nblk, 1, R)
    w1c = W1.astype(jnp.bfloat16)
    b1r = b1.reshape(1, H)
    b2r = b2.reshape(1, 1)

    pacc, pm, ps = pl.pallas_call(
        _stage1_kernel,
        grid=(nblk,),
        in_specs=[
            pl.BlockSpec((R, D), lambda i: (i, 0)),
            pl.BlockSpec((1, 1, R), lambda i: (i, 0, 0)),
            pl.BlockSpec((D, H), lambda i: (0, 0)),
            pl.BlockSpec((1, H), lambda i: (0, 0)),
            pl.BlockSpec((H, 1), lambda i: (0, 0)),
            pl.BlockSpec((1, 1), lambda i: (0, 0)),
        ],
        out_specs=[
            pl.BlockSpec((1, SEG, D), lambda i: (i, 0, 0)),
            pl.BlockSpec((1, SEG, 1), lambda i: (i, 0, 0)),
            pl.BlockSpec((1, SEG, 1), lambda i: (i, 0, 0)),
        ],
        out_shape=[
            jax.ShapeDtypeStruct((nblk, SEG, D), jnp.float32),
            jax.ShapeDtypeStruct((nblk, SEG, 1), jnp.float32),
            jax.ShapeDtypeStruct((nblk, SEG, 1), jnp.float32),
        ],
        compiler_params=pltpu.CompilerParams(
            dimension_semantics=("parallel",)),
    )(x, batch3, w1c, b1r, W2, b2r)

    out = pl.pallas_call(
        _stage2_kernel,
        grid=(nblk,),
        in_specs=[
            pl.BlockSpec((1, SEG, D), lambda i: (i, 0, 0)),
            pl.BlockSpec((1, SEG, 1), lambda i: (i, 0, 0)),
            pl.BlockSpec((1, SEG, 1), lambda i: (i, 0, 0)),
        ],
        out_specs=pl.BlockSpec((SEG, D), lambda i: (0, 0)),
        out_shape=jax.ShapeDtypeStruct((SEG, D), jnp.float32),
        scratch_shapes=[
            pltpu.VMEM((SEG, D), jnp.float32),
            pltpu.VMEM((SEG, 1), jnp.float32),
            pltpu.VMEM((SEG, 1), jnp.float32),
        ],
        compiler_params=pltpu.CompilerParams(
            dimension_semantics=("arbitrary",)),
    )(pacc, pm, ps)

    return out
